# initial kernel scaffold (unmeasured)
import jax
import jax.numpy as jnp
from jax import lax
from jax.experimental import pallas as pl
from jax.experimental.pallas import tpu as pltpu

_CompilerParams = (
    pltpu.CompilerParams
    if hasattr(pltpu, "CompilerParams")
    else pltpu.TPUCompilerParams
)


def kernel(Q, K, V):
    b, s_loc, h, d = Q.shape
    hd = h * d

    q = Q.astype(jnp.bfloat16).reshape(b, s_loc, hd)
    kv = jnp.concatenate(
        [
            K.astype(jnp.bfloat16).reshape(b, s_loc, hd),
            V.astype(jnp.bfloat16).reshape(b, s_loc, hd),
        ],
        axis=-1,
    )

    def comm_body(kv_ref, out_ref, send_sem, recv_sem):
        my_x = lax.axis_index("x")
        my_y = lax.axis_index("y")

        bsem = pltpu.get_barrier_semaphore()
        pl.semaphore_signal(
            bsem,
            inc=1,
            device_id=(my_x, 1 - my_y),
            device_id_type=pl.DeviceIdType.MESH,
        )
        pl.semaphore_wait(bsem, 1)

        @pl.when(my_y == 0)
        def _():
            out_ref[0] = kv_ref[...]
            rdma = pltpu.make_async_remote_copy(
                src_ref=out_ref.at[0],
                dst_ref=out_ref.at[0],
                send_sem=send_sem,
                recv_sem=recv_sem,
                device_id=(my_x, 1),
                device_id_type=pl.DeviceIdType.MESH,
            )
            rdma.start()
            rdma.wait()

        @pl.when(my_y == 1)
        def _():
            out_ref[1] = kv_ref[...]
            rdma = pltpu.make_async_remote_copy(
                src_ref=out_ref.at[1],
                dst_ref=out_ref.at[1],
                send_sem=send_sem,
                recv_sem=recv_sem,
                device_id=(my_x, 0),
                device_id_type=pl.DeviceIdType.MESH,
            )
            rdma.start()
            rdma.wait()

    gathered = pl.pallas_call(
        comm_body,
        out_shape=jax.ShapeDtypeStruct((2, b, s_loc, 2 * hd), jnp.bfloat16),
        in_specs=[pl.BlockSpec(memory_space=pltpu.VMEM)],
        out_specs=pl.BlockSpec(memory_space=pltpu.VMEM),
        scratch_shapes=[pltpu.SemaphoreType.DMA, pltpu.SemaphoreType.DMA],
        compiler_params=_CompilerParams(collective_id=0),
    )(kv)

    scale = d**-0.5

    def attn_body(q_ref, k_ref, v_ref, o_ref):
        qb = q_ref[0]
        k0 = k_ref[0, 0]
        k1 = k_ref[1, 0]
        v0 = v_ref[0, 0]
        v1 = v_ref[1, 0]
        dn = (((1,), (1,)), ((), ()))
        s0 = lax.dot_general(qb, k0, dn, preferred_element_type=jnp.float32)
        s1 = lax.dot_general(qb, k1, dn, preferred_element_type=jnp.float32)
        s = jnp.concatenate([s0, s1], axis=1) * scale
        m = jnp.max(s, axis=1, keepdims=True)
        p = jnp.exp(s - m)
        p = p / jnp.sum(p, axis=1, keepdims=True)
        pb = p.astype(jnp.bfloat16)
        dn2 = (((1,), (0,)), ((), ()))
        o = lax.dot_general(
            pb[:, :s_loc], v0, dn2, preferred_element_type=jnp.float32
        )
        o = o + lax.dot_general(
            pb[:, s_loc:], v1, dn2, preferred_element_type=jnp.float32
        )
        o_ref[0] = o

    out = pl.pallas_call(
        attn_body,
        grid=(b, h),
        out_shape=jax.ShapeDtypeStruct((b, s_loc, hd), jnp.float32),
        in_specs=[
            pl.BlockSpec((1, s_loc, d), lambda bi, hi: (bi, 0, hi)),
            pl.BlockSpec((2, 1, s_loc, d), lambda bi, hi: (0, bi, 0, hi)),
            pl.BlockSpec((2, 1, s_loc, d), lambda bi, hi: (0, bi, 0, h + hi)),
        ],
        out_specs=pl.BlockSpec((1, s_loc, d), lambda bi, hi: (bi, 0, hi)),
    )(q, gathered, gathered)

    return out.reshape(b, s_loc, h, d)


# baseline (device time: 101017 ns/iter reference)
import jax
import jax.numpy as jnp
from jax import lax
from jax.experimental import pallas as pl
from jax.experimental.pallas import tpu as pltpu

_CompilerParams = (
    pltpu.CompilerParams
    if hasattr(pltpu, "CompilerParams")
    else pltpu.TPUCompilerParams
)


def kernel(Q, K, V):
    b, s_loc, h, d = Q.shape
    hd = h * d

    q = Q.astype(jnp.bfloat16).reshape(b, s_loc, hd)
    kv = jnp.concatenate(
        [
            K.astype(jnp.bfloat16).reshape(b, s_loc, hd),
            V.astype(jnp.bfloat16).reshape(b, s_loc, hd),
        ],
        axis=-1,
    )

    def comm_body(kv_ref, out_ref, send_sem, recv_sem):
        my_x = lax.axis_index("x")
        my_y = lax.axis_index("y")

        bsem = pltpu.get_barrier_semaphore()
        pl.semaphore_signal(
            bsem,
            inc=1,
            device_id=(my_x, 1 - my_y),
            device_id_type=pl.DeviceIdType.MESH,
        )
        pl.semaphore_wait(bsem, 1)

        @pl.when(my_y == 0)
        def _():
            out_ref[0] = kv_ref[...]
            rdma = pltpu.make_async_remote_copy(
                src_ref=out_ref.at[0],
                dst_ref=out_ref.at[0],
                send_sem=send_sem,
                recv_sem=recv_sem,
                device_id=(my_x, 1),
                device_id_type=pl.DeviceIdType.MESH,
            )
            rdma.start()
            rdma.wait()

        @pl.when(my_y == 1)
        def _():
            out_ref[1] = kv_ref[...]
            rdma = pltpu.make_async_remote_copy(
                src_ref=out_ref.at[1],
                dst_ref=out_ref.at[1],
                send_sem=send_sem,
                recv_sem=recv_sem,
                device_id=(my_x, 0),
                device_id_type=pl.DeviceIdType.MESH,
            )
            rdma.start()
            rdma.wait()

    gathered = pl.pallas_call(
        comm_body,
        out_shape=jax.ShapeDtypeStruct((2, b, s_loc, 2 * hd), jnp.bfloat16),
        in_specs=[pl.BlockSpec(memory_space=pltpu.VMEM)],
        out_specs=pl.BlockSpec(memory_space=pltpu.VMEM),
        scratch_shapes=[pltpu.SemaphoreType.DMA, pltpu.SemaphoreType.DMA],
        compiler_params=_CompilerParams(collective_id=0),
    )(kv)

    scale = d**-0.5
    hp = h // 2

    def attn_body(q_ref, k_ref, v_ref, o_ref):
        qb = q_ref[0]
        k0 = k_ref[0, 0]
        k1 = k_ref[1, 0]
        v0 = v_ref[0, 0]
        v1 = v_ref[1, 0]
        dn = (((1,), (1,)), ((), ()))
        dn2 = (((1,), (0,)), ((), ()))
        outs = []
        for j in (0, 1):
            sl = slice(j * d, (j + 1) * d)
            qh = qb[:, sl]
            s0 = lax.dot_general(
                qh, k0[:, sl], dn, preferred_element_type=jnp.float32
            )
            s1 = lax.dot_general(
                qh, k1[:, sl], dn, preferred_element_type=jnp.float32
            )
            s = jnp.concatenate([s0, s1], axis=1) * scale
            m = jnp.max(s, axis=1, keepdims=True)
            p = jnp.exp(s - m)
            p = p / jnp.sum(p, axis=1, keepdims=True)
            pb = p.astype(jnp.bfloat16)
            o = lax.dot_general(
                pb[:, :s_loc], v0[:, sl], dn2,
                preferred_element_type=jnp.float32,
            )
            o = o + lax.dot_general(
                pb[:, s_loc:], v1[:, sl], dn2,
                preferred_element_type=jnp.float32,
            )
            outs.append(o)
        o_ref[0] = jnp.concatenate(outs, axis=1)

    out = pl.pallas_call(
        attn_body,
        grid=(b, hp),
        out_shape=jax.ShapeDtypeStruct((b, s_loc, hd), jnp.float32),
        in_specs=[
            pl.BlockSpec((1, s_loc, 2 * d), lambda bi, hi: (bi, 0, hi)),
            pl.BlockSpec((2, 1, s_loc, 2 * d), lambda bi, hi: (0, bi, 0, hi)),
            pl.BlockSpec(
                (2, 1, s_loc, 2 * d), lambda bi, hi: (0, bi, 0, hp + hi)
            ),
        ],
        out_specs=pl.BlockSpec((1, s_loc, 2 * d), lambda bi, hi: (bi, 0, hi)),
    )(q, gathered, gathered)

    return out.reshape(b, s_loc, h, d)


# device time: 72993 ns/iter; 1.3839x vs baseline; 1.3839x over previous
import jax
import jax.numpy as jnp
from jax import lax
from jax.experimental import pallas as pl
from jax.experimental.pallas import tpu as pltpu

_CompilerParams = (
    pltpu.CompilerParams
    if hasattr(pltpu, "CompilerParams")
    else pltpu.TPUCompilerParams
)

_MESH = pl.DeviceIdType.MESH
NC = 8


def kernel(Q, K, V):
    b, s_loc, h, d = Q.shape
    hd = h * d
    cs = s_loc // NC

    qs = Q.reshape(b, s_loc, hd)
    kb = K.astype(jnp.bfloat16).reshape(b, s_loc, hd)
    vb = V.astype(jnp.bfloat16).reshape(b, s_loc, hd)

    def comm_body(k_ref, v_ref, out_ref, ys_k, yr_k, ys_v, yr_v, xs, xr):
        my_x = lax.axis_index("x")
        my_y = lax.axis_index("y")
        ynbr = (my_x, 1 - my_y)
        xnbr = (1 - my_x, my_y)

        bsem = pltpu.get_barrier_semaphore()
        pl.semaphore_signal(bsem, inc=1, device_id=ynbr, device_id_type=_MESH)
        pl.semaphore_signal(bsem, inc=1, device_id=xnbr, device_id_type=_MESH)
        pl.semaphore_wait(bsem, 2)

        bh = pl.ds(2 * my_x, 2)
        ydk, ydv, xd = [], [], []
        for i in range(NC):
            sc = pl.ds(i * cs, cs)
            ydk.append(
                pltpu.make_async_remote_copy(
                    src_ref=k_ref.at[bh, sc],
                    dst_ref=out_ref.at[bh, sc, pl.ds(0, hd)],
                    send_sem=ys_k.at[i],
                    recv_sem=yr_k.at[i],
                    device_id=ynbr,
                    device_id_type=_MESH,
                )
            )
            ydv.append(
                pltpu.make_async_remote_copy(
                    src_ref=v_ref.at[bh, sc],
                    dst_ref=out_ref.at[bh, sc, pl.ds(hd, hd)],
                    send_sem=ys_v.at[i],
                    recv_sem=yr_v.at[i],
                    device_id=ynbr,
                    device_id_type=_MESH,
                )
            )
            xd.append(
                pltpu.make_async_remote_copy(
                    src_ref=out_ref.at[bh, sc],
                    dst_ref=out_ref.at[bh, sc],
                    send_sem=xs.at[i],
                    recv_sem=xr.at[i],
                    device_id=xnbr,
                    device_id_type=_MESH,
                )
            )
        for i in range(NC):
            ydk[i].start()
            ydv[i].start()
        for i in range(NC):
            ydk[i].wait_recv()
            ydv[i].wait_recv()
            xd[i].start()
        for i in range(NC):
            ydk[i].wait_send()
            ydv[i].wait_send()
        for i in range(NC):
            xd[i].wait_send()
            xd[i].wait_recv()

    gathered = pl.pallas_call(
        comm_body,
        out_shape=jax.ShapeDtypeStruct((b, s_loc, 2 * hd), jnp.bfloat16),
        in_specs=[
            pl.BlockSpec(memory_space=pltpu.VMEM),
            pl.BlockSpec(memory_space=pltpu.VMEM),
        ],
        out_specs=pl.BlockSpec(memory_space=pltpu.VMEM),
        scratch_shapes=[pltpu.SemaphoreType.DMA((NC,))] * 6,
        compiler_params=_CompilerParams(collective_id=0),
    )(kb, vb)

    scale = d**-0.5
    hp = h // 2

    def attn_body(q_ref, kl_ref, vl_ref, kr_ref, vr_ref, o_ref):
        ones = jnp.ones((s_loc, d), jnp.bfloat16)
        dn = (((1,), (1,)), ((), ()))
        dn2 = (((1,), (0,)), ((), ()))
        outs = []
        for j in (0, 1):
            sl = slice(j * d, (j + 1) * d)
            qh = (q_ref[0][:, sl] * scale).astype(jnp.bfloat16)
            s_l = lax.dot_general(
                qh, kl_ref[0][:, sl], dn, preferred_element_type=jnp.float32
            )
            s_r = lax.dot_general(
                qh, kr_ref[0][:, sl], dn, preferred_element_type=jnp.float32
            )
            p_l = jnp.exp(s_l).astype(jnp.bfloat16)
            p_r = jnp.exp(s_r).astype(jnp.bfloat16)
            vx_l = jnp.concatenate([vl_ref[0][:, sl], ones], axis=1)
            vx_r = jnp.concatenate([vr_ref[0][:, sl], ones], axis=1)
            r = lax.dot_general(
                p_l, vx_l, dn2, preferred_element_type=jnp.float32
            ) + lax.dot_general(
                p_r, vx_r, dn2, preferred_element_type=jnp.float32
            )
            outs.append(r[:, :d] / r[:, d:])
        o_ref[0] = jnp.concatenate(outs, axis=1)

    out = pl.pallas_call(
        attn_body,
        grid=(b, hp),
        out_shape=jax.ShapeDtypeStruct((b, s_loc, hd), jnp.float32),
        in_specs=[
            pl.BlockSpec((1, s_loc, 2 * d), lambda bi, hi: (bi, 0, hi)),
            pl.BlockSpec((1, s_loc, 2 * d), lambda bi, hi: (bi, 0, hi)),
            pl.BlockSpec((1, s_loc, 2 * d), lambda bi, hi: (bi, 0, hi)),
            pl.BlockSpec((1, s_loc, 2 * d), lambda bi, hi: (bi, 0, hi)),
            pl.BlockSpec((1, s_loc, 2 * d), lambda bi, hi: (bi, 0, hp + hi)),
        ],
        out_specs=pl.BlockSpec((1, s_loc, 2 * d), lambda bi, hi: (bi, 0, hi)),
    )(qs, kb, vb, gathered, gathered)

    return out.reshape(b, s_loc, h, d)


# device time: 47627 ns/iter; 2.1210x vs baseline; 1.5326x over previous
import jax
import jax.numpy as jnp
from jax import lax
from jax.experimental import pallas as pl
from jax.experimental.pallas import tpu as pltpu

_CompilerParams = (
    pltpu.CompilerParams
    if hasattr(pltpu, "CompilerParams")
    else pltpu.TPUCompilerParams
)

_MESH = pl.DeviceIdType.MESH
NC = 8
_LOG2E = 1.4426950408889634


def kernel(Q, K, V):
    b, s_loc, h, d = Q.shape
    hd = h * d
    hp = h // 2
    cs = s_loc // (NC // 2)
    scale = d**-0.5 * _LOG2E

    qs = Q.reshape(b, s_loc, hd)
    kb = K.astype(jnp.bfloat16).reshape(b, s_loc, hd)
    vb = V.astype(jnp.bfloat16).reshape(b, s_loc, hd)

    def body(q_ref, k_ref, v_ref, o_ref, rkv, racc,
             yks, ykr, yvs, yvr, xs, xr):
        my_x = lax.axis_index("x")
        my_y = lax.axis_index("y")
        ynbr = (my_x, 1 - my_y)
        xnbr = (1 - my_x, my_y)
        mb = 2 * my_x
        ob = 2 * (1 - my_x)

        bsem = pltpu.get_barrier_semaphore()
        pl.semaphore_signal(bsem, inc=1, device_id=ynbr, device_id_type=_MESH)
        pl.semaphore_signal(bsem, inc=1, device_id=xnbr, device_id_type=_MESH)
        pl.semaphore_wait(bsem, 2)

        ydk, ydv, xd = [], [], []
        for c in range(NC):
            bsl = pl.ds(mb + c // (NC // 2), 1)
            ssl = pl.ds((c % (NC // 2)) * cs, cs)
            ydk.append(
                pltpu.make_async_remote_copy(
                    src_ref=k_ref.at[bsl, ssl],
                    dst_ref=rkv.at[bsl, ssl, pl.ds(0, hd)],
                    send_sem=yks.at[c],
                    recv_sem=ykr.at[c],
                    device_id=ynbr,
                    device_id_type=_MESH,
                )
            )
            ydv.append(
                pltpu.make_async_remote_copy(
                    src_ref=v_ref.at[bsl, ssl],
                    dst_ref=rkv.at[bsl, ssl, pl.ds(hd, hd)],
                    send_sem=yvs.at[c],
                    recv_sem=yvr.at[c],
                    device_id=ynbr,
                    device_id_type=_MESH,
                )
            )
            xd.append(
                pltpu.make_async_remote_copy(
                    src_ref=rkv.at[bsl, ssl],
                    dst_ref=rkv.at[bsl, ssl],
                    send_sem=xs.at[c],
                    recv_sem=xr.at[c],
                    device_id=xnbr,
                    device_id_type=_MESH,
                )
            )
        for c in range(NC):
            ydk[c].start()
            ydv[c].start()

        ones = jnp.ones((s_loc, d), jnp.bfloat16)
        dn = (((1,), (1,)), ((), ()))
        dn2 = (((1,), (0,)), ((), ()))

        def qh_of(bi, hi, j):
            qp = q_ref[bi, :, hi * 2 * d:(hi + 1) * 2 * d]
            return (qp[:, j * d:(j + 1) * d] * scale).astype(jnp.bfloat16)

        def partial(qh, kk, vv):
            s = lax.dot_general(qh, kk, dn, preferred_element_type=jnp.float32)
            p = jnp.exp2(s).astype(jnp.bfloat16)
            vx = jnp.concatenate([vv, ones], axis=1)
            return lax.dot_general(p, vx, dn2,
                                   preferred_element_type=jnp.float32)

        blocks = [(bi, hi) for bi in range(b) for hi in range(hp)]
        per = max(1, len(blocks) // NC)
        for c in range(NC):
            for bi, hi in blocks[c * per:(c + 1) * per]:
                rs = [partial(qh_of(bi, hi, j),
                              k_ref[bi, :, hi * 2 * d:(hi + 1) * 2 * d]
                              [:, j * d:(j + 1) * d],
                              v_ref[bi, :, hi * 2 * d:(hi + 1) * 2 * d]
                              [:, j * d:(j + 1) * d])
                      for j in (0, 1)]
                racc[bi, hi] = jnp.concatenate(rs, axis=1)
            ydk[c].wait_recv()
            ydv[c].wait_recv()
            xd[c].start()
        for bi, hi in blocks[NC * per:]:
            rs = [partial(qh_of(bi, hi, j),
                          k_ref[bi, :, hi * 2 * d:(hi + 1) * 2 * d]
                          [:, j * d:(j + 1) * d],
                          v_ref[bi, :, hi * 2 * d:(hi + 1) * 2 * d]
                          [:, j * d:(j + 1) * d])
                  for j in (0, 1)]
            racc[bi, hi] = jnp.concatenate(rs, axis=1)

        for c in range(NC):
            ydk[c].wait_send()
            ydv[c].wait_send()

        for half in (0, 1):
            if half == 1:
                for c in range(NC):
                    xd[c].wait_send()
                    xd[c].wait_recv()
            for bj in (0, 1):
                bi = (mb if half == 0 else ob) + bj
                for hi in range(hp):
                    kv_pair = rkv[bi, :, :]
                    outs = []
                    for j in (0, 1):
                        off = hi * 2 * d + j * d
                        kk = kv_pair[:, off:off + d]
                        vv = kv_pair[:, hd + off:hd + off + d]
                        rr = partial(qh_of(bi, hi, j), kk, vv)
                        rt = racc[bi, hi, :, j * 2 * d:(j + 1) * 2 * d] + rr
                        outs.append(rt[:, :d] / rt[:, d:])
                    o_ref[bi, :, hi * 2 * d:(hi + 1) * 2 * d] = (
                        jnp.concatenate(outs, axis=1)
                    )

    out = pl.pallas_call(
        body,
        out_shape=jax.ShapeDtypeStruct((b, s_loc, hd), jnp.float32),
        in_specs=[pl.BlockSpec(memory_space=pltpu.VMEM)] * 3,
        out_specs=pl.BlockSpec(memory_space=pltpu.VMEM),
        scratch_shapes=[
            pltpu.VMEM((b, s_loc, 2 * hd), jnp.bfloat16),
            pltpu.VMEM((b, hp, s_loc, 4 * d), jnp.float32),
            pltpu.SemaphoreType.DMA((NC,)),
            pltpu.SemaphoreType.DMA((NC,)),
            pltpu.SemaphoreType.DMA((NC,)),
            pltpu.SemaphoreType.DMA((NC,)),
            pltpu.SemaphoreType.DMA((NC,)),
            pltpu.SemaphoreType.DMA((NC,)),
        ],
        compiler_params=_CompilerParams(collective_id=0),
    )(qs, kb, vb)

    return out.reshape(b, s_loc, h, d)


# device time: 45709 ns/iter; 2.2100x vs baseline; 1.0420x over previous
import jax
import jax.numpy as jnp
from jax import lax
from jax.experimental import pallas as pl
from jax.experimental.pallas import tpu as pltpu

_CompilerParams = (
    pltpu.CompilerParams
    if hasattr(pltpu, "CompilerParams")
    else pltpu.TPUCompilerParams
)

_MESH = pl.DeviceIdType.MESH
NC = 8
_LOG2E = 1.4426950408889634


def kernel(Q, K, V):
    b, s_loc, h, d = Q.shape
    hd = h * d
    hp = h // 2
    nq = NC // 2
    cs = s_loc // nq
    scale = d**-0.5 * _LOG2E

    qb = (Q * scale).astype(jnp.bfloat16).reshape(b, s_loc, hd)
    kb = K.astype(jnp.bfloat16).reshape(b, s_loc, hd)
    vb = V.astype(jnp.bfloat16).reshape(b, s_loc, hd)

    def body(q_ref, k_ref, v_ref, o_ref, rkv, racc,
             yks, ykr, yvs, yvr, xs, xr):
        my_x = lax.axis_index("x")
        my_y = lax.axis_index("y")
        ynbr = (my_x, 1 - my_y)
        xnbr = (1 - my_x, my_y)
        mb = 2 * my_x
        ob = 2 * (1 - my_x)

        bsem = pltpu.get_barrier_semaphore()
        pl.semaphore_signal(bsem, inc=1, device_id=ynbr, device_id_type=_MESH)
        pl.semaphore_signal(bsem, inc=1, device_id=xnbr, device_id_type=_MESH)
        pl.semaphore_wait(bsem, 2)

        ydk, ydv, xd = [], [], []
        for c in range(NC):
            bsl = pl.ds(mb + c // nq, 1)
            ssl = pl.ds((c % nq) * cs, cs)
            ydk.append(
                pltpu.make_async_remote_copy(
                    src_ref=k_ref.at[bsl, ssl],
                    dst_ref=rkv.at[bsl, ssl, pl.ds(0, hd)],
                    send_sem=yks.at[c],
                    recv_sem=ykr.at[c],
                    device_id=ynbr,
                    device_id_type=_MESH,
                )
            )
            ydv.append(
                pltpu.make_async_remote_copy(
                    src_ref=v_ref.at[bsl, ssl],
                    dst_ref=rkv.at[bsl, ssl, pl.ds(hd, hd)],
                    send_sem=yvs.at[c],
                    recv_sem=yvr.at[c],
                    device_id=ynbr,
                    device_id_type=_MESH,
                )
            )
            xd.append(
                pltpu.make_async_remote_copy(
                    src_ref=rkv.at[bsl, ssl],
                    dst_ref=rkv.at[bsl, ssl],
                    send_sem=xs.at[c],
                    recv_sem=xr.at[c],
                    device_id=xnbr,
                    device_id_type=_MESH,
                )
            )
        for c in range(NC):
            ydk[c].start()
            ydv[c].start()

        ones = jnp.ones((s_loc, d), jnp.bfloat16)
        dn = (((1,), (1,)), ((), ()))
        dn2 = (((1,), (0,)), ((), ()))

        def partial(qh, kk, vv):
            s = lax.dot_general(qh, kk, dn, preferred_element_type=jnp.float32)
            p = jnp.exp2(s).astype(jnp.bfloat16)
            vx = jnp.concatenate([vv, ones], axis=1)
            return lax.dot_general(p, vx, dn2,
                                   preferred_element_type=jnp.float32)

        def local_block(bi, hi):
            qp = q_ref[bi, :, hi * 2 * d:(hi + 1) * 2 * d]
            kp = k_ref[bi, :, hi * 2 * d:(hi + 1) * 2 * d]
            vp = v_ref[bi, :, hi * 2 * d:(hi + 1) * 2 * d]
            rs = [partial(qp[:, j * d:(j + 1) * d],
                          kp[:, j * d:(j + 1) * d],
                          vp[:, j * d:(j + 1) * d]) for j in (0, 1)]
            racc[bi, hi] = jnp.concatenate(rs, axis=1)

        def remote_block(bi, hi):
            qp = q_ref[bi, :, hi * 2 * d:(hi + 1) * 2 * d]
            kv_pair = rkv[bi, :, :]
            outs = []
            for j in (0, 1):
                off = hi * 2 * d + j * d
                rr = partial(qp[:, j * d:(j + 1) * d],
                             kv_pair[:, off:off + d],
                             kv_pair[:, hd + off:hd + off + d])
                rt = racc[bi, hi, :, j * 2 * d:(j + 1) * 2 * d] + rr
                outs.append(rt[:, :d] / rt[:, d:])
            o_ref[bi, :, hi * 2 * d:(hi + 1) * 2 * d] = (
                jnp.concatenate(outs, axis=1)
            )

        blocks = [(bi, hi) for bi in range(b) for hi in range(hp)]
        for c in range(nq):
            for bi, hi in blocks[c * 4:(c + 1) * 4]:
                local_block(bi, hi)
            ydk[c].wait_recv()
            ydv[c].wait_recv()
            xd[c].start()
        for c in range(nq, NC):
            for bi, hi in blocks[c * 4:(c + 1) * 4]:
                local_block(bi, hi)
            for hi in (2 * (c - nq), 2 * (c - nq) + 1):
                remote_block(mb, hi)
            ydk[c].wait_recv()
            ydv[c].wait_recv()
            xd[c].start()
        for hi in range(hp):
            remote_block(mb + 1, hi)
        for c in range(NC):
            ydk[c].wait_send()
            ydv[c].wait_send()
        for bj in (0, 1):
            for c in range(bj * nq, (bj + 1) * nq):
                xd[c].wait_send()
                xd[c].wait_recv()
            for hi in range(hp):
                remote_block(ob + bj, hi)

    out = pl.pallas_call(
        body,
        out_shape=jax.ShapeDtypeStruct((b, s_loc, hd), jnp.float32),
        in_specs=[pl.BlockSpec(memory_space=pltpu.VMEM)] * 3,
        out_specs=pl.BlockSpec(memory_space=pltpu.VMEM),
        scratch_shapes=[
            pltpu.VMEM((b, s_loc, 2 * hd), jnp.bfloat16),
            pltpu.VMEM((b, hp, s_loc, 4 * d), jnp.float32),
            pltpu.SemaphoreType.DMA((NC,)),
            pltpu.SemaphoreType.DMA((NC,)),
            pltpu.SemaphoreType.DMA((NC,)),
            pltpu.SemaphoreType.DMA((NC,)),
            pltpu.SemaphoreType.DMA((NC,)),
            pltpu.SemaphoreType.DMA((NC,)),
        ],
        compiler_params=_CompilerParams(collective_id=0),
    )(qb, kb, vb)

    return out.reshape(b, s_loc, h, d)
